# baseline (speedup 1.0000x reference)
import functools

import jax
import jax.numpy as jnp
from jax.experimental import pallas as pl
from jax.experimental.pallas import tpu as pltpu

COMPRESSION_RATIO = 0.8
MAX_TOKENS = 2048

TN = 512


def _scores_kernel(bag_ref, wkt_ref, bk_ref, tq_ref, out_ref):
    pk = jnp.dot(bag_ref[0], wkt_ref[...], preferred_element_type=jnp.float32)
    pk = pk + bk_ref[...]
    s = jnp.dot(pk, tq_ref[0].reshape(-1, 1), preferred_element_type=jnp.float32)
    out_ref[...] = s.reshape(1, 1, -1)


def _scores(padded_bag, W_k_T, b_k, text_q):
    B, N, D = padded_bag.shape
    grid = (B, N // TN)
    out = pl.pallas_call(
        _scores_kernel,
        grid=grid,
        in_specs=[
            pl.BlockSpec((1, TN, D), lambda b, n: (b, n, 0)),
            pl.BlockSpec((D, D), lambda b, n: (0, 0)),
            pl.BlockSpec((1, D), lambda b, n: (0, 0)),
            pl.BlockSpec((1, 1, D), lambda b, n: (b, 0, 0)),
        ],
        out_specs=pl.BlockSpec((1, 1, TN), lambda b, n: (b * (N // TN) + n, 0, 0)),
        out_shape=jax.ShapeDtypeStruct((B * (N // TN), 1, TN), jnp.float32),
    )(padded_bag, W_k_T, b_k, text_q.reshape(B, 1, D))
    return out.reshape(B, N)


def kernel(padded_bag, key_padding_mask, text_feature_batch, W_q, b_q, W_k, b_k):
    B, N, D = padded_bag.shape
    num_patches = (~key_padding_mask).sum(axis=1)
    k_per_bag = (num_patches.astype(jnp.float32) * COMPRESSION_RATIO).astype(jnp.int32)
    k_per_bag = jnp.clip(k_per_bag, 1, MAX_TOKENS)
    k_per_bag = jnp.minimum(k_per_bag, num_patches.astype(jnp.int32))
    k_per_bag = jnp.where(k_per_bag == 0, 1, k_per_bag)
    max_k = min(max(1, min(int(N * COMPRESSION_RATIO), MAX_TOKENS)), N)

    text_q = text_feature_batch @ W_q.T + b_q
    scores = _scores(padded_bag, W_k.T, b_k.reshape(1, D), text_q)
    scores = jnp.where(key_padding_mask, -jnp.inf, scores)
    _, idx = jax.lax.top_k(scores, max_k)
    compressed = jnp.take_along_axis(padded_bag, idx[:, :, None], axis=1)
    new_mask = jnp.arange(max_k)[None, :] >= k_per_bag[:, None]
    return (compressed, new_mask)
